# baseline (speedup 1.0000x reference)
import functools

import jax
import jax.numpy as jnp
import numpy as np
from jax import lax
from jax.experimental import pallas as pl
from jax.experimental.pallas import tpu as pltpu
from jax.experimental.pallas import tpu_sc as plsc

N = 10000
E = 640000
T = 1280000
KMAX = 4
NSPH = 7
DA = 128
DE = 64
DRBF = 16
DCBF = 16
DT = 32
INV_SQRT_2 = float(1.0 / np.sqrt(2.0))


def _silu(x):
    return x * jax.nn.sigmoid(x)


def _tc1_body(m_st_ref, rbf_ref, wmkt_ref, wrbf3_ref, wdown_ref, out_ref):
    mkt = _silu(m_st_ref[...] @ wmkt_ref[...])
    mkt = mkt * (rbf_ref[...] @ wrbf3_ref[...])
    out_ref[...] = _silu(mkt @ wdown_ref[...])


def _tc1(m_st, rbf, W_mkt, W_rbf3, W_down, be=8000):
    grid = (E // be,)
    return pl.pallas_call(
        _tc1_body,
        grid=grid,
        in_specs=[
            pl.BlockSpec((be, DE), lambda i: (i, 0)),
            pl.BlockSpec((be, DRBF), lambda i: (i, 0)),
            pl.BlockSpec((DE, DE), lambda i: (0, 0)),
            pl.BlockSpec((DRBF, DE), lambda i: (0, 0)),
            pl.BlockSpec((DE, DT), lambda i: (0, 0)),
        ],
        out_specs=pl.BlockSpec((be, DT), lambda i: (i, 0)),
        out_shape=jax.ShapeDtypeStruct((E, DT), jnp.float32),
    )(m_st, rbf, W_mkt, W_rbf3, W_down)


SC_LANES = NSPH * DT
RW_LANES = DCBF * DT


def _routing_consts():
    P = np.zeros((KMAX * NSPH, KMAX * SC_LANES), np.float32)
    Q = np.zeros((KMAX * DT, KMAX * SC_LANES), np.float32)
    for k in range(KMAX):
        for s in range(NSPH):
            for c in range(DT):
                P[k * NSPH + s, k * SC_LANES + s * DT + c] = 1.0
                Q[k * DT + c, k * SC_LANES + s * DT + c] = 1.0
    R = np.zeros((NSPH * DCBF * NSPH, RW_LANES), np.float32)
    Tm = np.zeros((NSPH * SC_LANES, RW_LANES), np.float32)
    for s in range(NSPH):
        for i in range(DCBF):
            for c in range(DT):
                R[s * (DCBF * NSPH) + i * NSPH + s, i * DT + c] = 1.0
                Tm[s * SC_LANES + s * DT + c, i * DT + c] = 1.0
    return jnp.asarray(P), jnp.asarray(Q), jnp.asarray(R), jnp.asarray(Tm)


def _tc3_body(m2f_ref, cbf1f_ref, cbf0f_ref, p_ref, q_ref, r_ref, t_ref,
              wf_ref, wst3_ref, wts3_ref, xst_ref, xts_ref):
    m2f = m2f_ref[...]
    cbf1f = cbf1f_ref[...]
    cbf0f = cbf0f_ref[...]

    cp = cbf1f @ p_ref[...]
    qp = m2f @ q_ref[...]
    sk = cp[:, :SC_LANES] * qp[:, :SC_LANES]
    for k in range(1, KMAX):
        sk += cp[:, k * SC_LANES:(k + 1) * SC_LANES] * qp[:, k * SC_LANES:(k + 1) * SC_LANES]

    rw = None
    for s in range(NSPH):
        r_s = cbf0f @ r_ref[s * (DCBF * NSPH):(s + 1) * (DCBF * NSPH), :]
        t_s = sk @ t_ref[s * SC_LANES:(s + 1) * SC_LANES, :]
        term = r_s * t_s
        rw = term if rw is None else rw + term

    x = rw @ wf_ref[...]
    xst_ref[...] = _silu(x @ wst3_ref[...])
    xts_ref[...] = _silu(x @ wts3_ref[...])


def _tc3(m2f, cbf1f, cbf0f, P, Q, R, Tm, Wf, W_st3, W_ts3, be=640):
    grid = (E // be,)
    out_shape = [
        jax.ShapeDtypeStruct((E, DE), jnp.float32),
        jax.ShapeDtypeStruct((E, DE), jnp.float32),
    ]
    return pl.pallas_call(
        _tc3_body,
        grid=grid,
        in_specs=[
            pl.BlockSpec((be, KMAX * DT), lambda i: (i, 0)),
            pl.BlockSpec((be, KMAX * NSPH), lambda i: (i, 0)),
            pl.BlockSpec((be, DCBF * NSPH), lambda i: (i, 0)),
            pl.BlockSpec(P.shape, lambda i: (0, 0)),
            pl.BlockSpec(Q.shape, lambda i: (0, 0)),
            pl.BlockSpec(R.shape, lambda i: (0, 0)),
            pl.BlockSpec(Tm.shape, lambda i: (0, 0)),
            pl.BlockSpec((RW_LANES, DT), lambda i: (0, 0)),
            pl.BlockSpec((DT, DE), lambda i: (0, 0)),
            pl.BlockSpec((DT, DE), lambda i: (0, 0)),
        ],
        out_specs=[
            pl.BlockSpec((be, DE), lambda i: (i, 0)),
            pl.BlockSpec((be, DE), lambda i: (i, 0)),
        ],
        out_shape=out_shape,
    )(m2f, cbf1f, cbf0f, P, Q, R, Tm, Wf, W_st3, W_ts3)


def _tc5_body(xst_ref, xts_ref, m_st_ref, rbf_ref, wca_ref, rbw1_ref, rbw2_ref,
              raw1_ref, raw2_ref, wrbfh_ref, m_ref, aout_ref):
    x_ca = _silu(m_st_ref[...] @ wca_ref[...])
    x = x_ca * INV_SQRT_2 + (xst_ref[...] + xts_ref[...]) * 0.5
    y = _silu(x @ rbw1_ref[...])
    y = _silu(y @ rbw2_ref[...])
    x = (x + y) * INV_SQRT_2
    m = (m_st_ref[...] + x) * INV_SQRT_2
    y = _silu(m @ raw1_ref[...])
    y = _silu(y @ raw2_ref[...])
    m = (m + y) * INV_SQRT_2
    m_ref[...] = m
    aout_ref[...] = m * (rbf_ref[...] @ wrbfh_ref[...])


def _tc5(x_st, xts_g, m_st, rbf, W_ca, rb_w1, rb_w2, ra_w1, ra_w2, W_rbf_h, be=8000):
    grid = (E // be,)
    return pl.pallas_call(
        _tc5_body,
        grid=grid,
        in_specs=[
            pl.BlockSpec((be, DE), lambda i: (i, 0)),
            pl.BlockSpec((be, DE), lambda i: (i, 0)),
            pl.BlockSpec((be, DE), lambda i: (i, 0)),
            pl.BlockSpec((be, DRBF), lambda i: (i, 0)),
            pl.BlockSpec((DE, DE), lambda i: (0, 0)),
            pl.BlockSpec((DE, DE), lambda i: (0, 0)),
            pl.BlockSpec((DE, DE), lambda i: (0, 0)),
            pl.BlockSpec((DE, DE), lambda i: (0, 0)),
            pl.BlockSpec((DE, DE), lambda i: (0, 0)),
            pl.BlockSpec((DRBF, DE), lambda i: (0, 0)),
        ],
        out_specs=[
            pl.BlockSpec((be, DE), lambda i: (i, 0)),
            pl.BlockSpec((be, DE), lambda i: (i, 0)),
        ],
        out_shape=[
            jax.ShapeDtypeStruct((E, DE), jnp.float32),
            jax.ShapeDtypeStruct((E, DE), jnp.float32),
        ],
    )(x_st, xts_g, m_st, rbf, W_ca, rb_w1, rb_w2, ra_w1, ra_w2, W_rbf_h)


def _tc7_body(p0_ref, p1_ref, h_ref, watom_ref, atw1_ref, atw2_ref,
              ws1_ref, ws2_ref, hnew_ref, hs1_ref, hs2_ref):
    x = p0_ref[...] + p1_ref[...]
    x = _silu(x @ watom_ref[...])
    y = _silu(x @ atw1_ref[...])
    y = _silu(y @ atw2_ref[...])
    x = (x + y) * INV_SQRT_2
    h_new = (h_ref[...] + x) * INV_SQRT_2
    hnew_ref[...] = h_new
    hs1_ref[...] = h_new @ ws1_ref[...]
    hs2_ref[...] = h_new @ ws2_ref[...]


def _tc7(p0, p1, h, W_atom, at_w1, at_w2, Ws1, Ws2, bn=2000):
    grid = (N // bn,)
    return pl.pallas_call(
        _tc7_body,
        grid=grid,
        in_specs=[
            pl.BlockSpec((bn, DE), lambda i: (i, 0)),
            pl.BlockSpec((bn, DE), lambda i: (i, 0)),
            pl.BlockSpec((bn, DA), lambda i: (i, 0)),
            pl.BlockSpec((DE, DA), lambda i: (0, 0)),
            pl.BlockSpec((DA, DA), lambda i: (0, 0)),
            pl.BlockSpec((DA, DA), lambda i: (0, 0)),
            pl.BlockSpec((DA, DE), lambda i: (0, 0)),
            pl.BlockSpec((DA, DE), lambda i: (0, 0)),
        ],
        out_specs=[
            pl.BlockSpec((bn, DA), lambda i: (i, 0)),
            pl.BlockSpec((bn, DE), lambda i: (i, 0)),
            pl.BlockSpec((bn, DE), lambda i: (i, 0)),
        ],
        out_shape=[
            jax.ShapeDtypeStruct((N, DA), jnp.float32),
            jax.ShapeDtypeStruct((N, DE), jnp.float32),
            jax.ShapeDtypeStruct((N, DE), jnp.float32),
        ],
    )(p0, p1, h, W_atom, at_w1, at_w2, Ws1, Ws2)


def _tc9_body(g1_ref, g2_ref, m_ref, ws3_ref, rsw1_ref, rsw2_ref, out_ref):
    m = m_ref[...]
    y = _silu(g1_ref[...] + g2_ref[...] + m @ ws3_ref[...])
    m = (m + y) * INV_SQRT_2
    z = _silu(m @ rsw1_ref[...])
    z = _silu(z @ rsw2_ref[...])
    out_ref[...] = (m + z) * INV_SQRT_2


def _tc9(g1, g2, m_mid, Ws3, rs_w1, rs_w2, be=8000):
    grid = (E // be,)
    return pl.pallas_call(
        _tc9_body,
        grid=grid,
        in_specs=[
            pl.BlockSpec((be, DE), lambda i: (i, 0)),
            pl.BlockSpec((be, DE), lambda i: (i, 0)),
            pl.BlockSpec((be, DE), lambda i: (i, 0)),
            pl.BlockSpec((DE, DE), lambda i: (0, 0)),
            pl.BlockSpec((DE, DE), lambda i: (0, 0)),
            pl.BlockSpec((DE, DE), lambda i: (0, 0)),
        ],
        out_specs=pl.BlockSpec((be, DE), lambda i: (i, 0)),
        out_shape=jax.ShapeDtypeStruct((E, DE), jnp.float32),
    )(g1, g2, m_mid, Ws3, rs_w1, rs_w2)


NW = 32
NC = 2
SLOTS = E * KMAX
SLOTS_W = SLOTS // NW
PADROWS = 64
CH3 = 2048
GK = 5
GSTREAM = 80
GROWS = GK * GSTREAM


def _sc_route_body(st_hbm, rg_hbm, kt_hbm, mkt_hbm, m2_hbm,
                   src_v, st_c, rg_c, kt_c, rows_v, sem, semw):
    wid = lax.axis_index("s") * NC + lax.axis_index("c")
    base_slot = wid * SLOTS_W
    lane = lax.iota(jnp.int32, 16)
    intmax = jnp.int32(2**31 - 1)
    nchunks = T // CH3

    def init_body(j, _):
        src_v[pl.ds(j * 16, 16)] = E + ((j * 16 + lane) & (PADROWS - 1))
        return 0
    lax.fori_loop(0, SLOTS_W // 16, init_body, 0)

    def prefetch(cidx, par):
        pltpu.async_copy(st_hbm.at[pl.ds(cidx * CH3, CH3)],
                         st_c.at[pl.ds(par * CH3, CH3)], sem)
        pltpu.async_copy(rg_hbm.at[pl.ds(cidx * CH3, CH3)],
                         rg_c.at[pl.ds(par * CH3, CH3)], sem)
        pltpu.async_copy(kt_hbm.at[pl.ds(cidx * CH3, CH3)],
                         kt_c.at[pl.ds(par * CH3, CH3)], sem)

    def drain(par):
        for buf in (st_c, rg_c, kt_c):
            pltpu.make_async_copy(st_hbm.at[pl.ds(0, CH3)],
                                  buf.at[pl.ds(par * CH3, CH3)], sem).wait()

    prefetch(0, 0)

    def chunk_body(cidx, _):
        par = lax.rem(cidx, 2)
        drain(par)

        @pl.when(cidx + 1 < nchunks)
        def _():
            prefetch(cidx + 1, 1 - par)

        def scan_one(v16):
            st = st_c[pl.ds(par * CH3 + v16, 16)]
            rg = rg_c[pl.ds(par * CH3 + v16, 16)]
            kt = kt_c[pl.ds(par * CH3 + v16, 16)]
            loc = st * KMAX + rg - base_slot
            inr = (loc >= 0) & (loc < SLOTS_W)
            nact = jnp.max(jnp.cumsum(jnp.where(inr, 1, 0)))

            @pl.when(nact == 1)
            def _():
                plsc.store_scatter(src_v, [jnp.where(inr, loc, 0)], kt, mask=inr)

            @pl.when(nact > 1)
            def _():
                comp = jnp.where(inr, loc * 16 + lane, intmax)
                ck, cv = plsc.sort_key_val(comp, kt)
                slot = lax.shift_right_arithmetic(ck, 4)
                nxt = ck.at[jnp.minimum(lane + 1, 15)].get(
                    mode="promise_in_bounds")
                win = (ck != intmax) & ((slot != lax.shift_right_arithmetic(nxt, 4))
                                        | (lane == 15))
                plsc.store_scatter(src_v, [jnp.where(win, slot, 0)], cv, mask=win)

        def vreg_body(v, _):
            scan_one(v * 32)
            scan_one(v * 32 + 16)
            return 0
        lax.fori_loop(0, CH3 // 32, vreg_body, 0)
        return 0
    lax.fori_loop(0, nchunks, chunk_body, 0)

    ngroups = SLOTS_W // GROWS

    def gather_body(g, _):
        par = lax.rem(g, 2)
        off = g * GROWS

        @pl.when(g >= 2)
        def _():
            pltpu.make_async_copy(
                rows_v.at[pl.ds(par * GROWS, GROWS)],
                m2_hbm.at[pl.ds(base_slot + (g - 2) * GROWS, GROWS)],
                semw).wait()

        descs = []
        for i in range(GK):
            descs.append(pltpu.async_copy(
                mkt_hbm.at[src_v.at[pl.ds(off + i * GSTREAM, GSTREAM)]],
                rows_v.at[pl.ds(par * GROWS + i * GSTREAM, GSTREAM)], sem))
        for d in descs:
            d.wait()
        pltpu.async_copy(rows_v.at[pl.ds(par * GROWS, GROWS)],
                         m2_hbm.at[pl.ds(base_slot + off, GROWS)], semw)
        return 0
    lax.fori_loop(0, ngroups, gather_body, 0)
    for gl in (ngroups - 2, ngroups - 1):
        pltpu.make_async_copy(
            rows_v.at[pl.ds(lax.rem(gl, 2) * GROWS, GROWS)],
            m2_hbm.at[pl.ds(base_slot + gl * GROWS, GROWS)], semw).wait()


def _sc_route(id3_st, id3_rg, id3_kt, mkt_pad):
    mesh = plsc.VectorSubcoreMesh(core_axis_name="c", subcore_axis_name="s")
    f = pl.kernel(
        _sc_route_body,
        out_type=jax.ShapeDtypeStruct((SLOTS, DT), jnp.float32),
        mesh=mesh,
        compiler_params=pltpu.CompilerParams(use_tc_tiling_on_sc=False, needs_layout_passes=False),
        scratch_types=[
            pltpu.VMEM((SLOTS_W,), jnp.int32),
            pltpu.VMEM((2 * CH3,), jnp.int32),
            pltpu.VMEM((2 * CH3,), jnp.int32),
            pltpu.VMEM((2 * CH3,), jnp.int32),
            pltpu.VMEM((2 * GROWS, DT), jnp.float32),
            pltpu.SemaphoreType.DMA,
            pltpu.SemaphoreType.DMA,
        ],
    )
    return f(id3_st, id3_rg, id3_kt, mkt_pad)


ROWS_W = E // NW
GG = 4
GGROWS = GG * 128
NFULL = ROWS_W // 128
TAIL = ROWS_W - NFULL * 128


def _gather_worker(tbl_hbm, idx_hbm, out_hbm, idx_c, rows_v, sem, wbase):
    def group_body(g, _):
        off = wbase + g * GGROWS
        pltpu.sync_copy(idx_hbm.at[pl.ds(off, GGROWS)], idx_c)
        descs = []
        for i in range(GG):
            descs.append(pltpu.async_copy(
                tbl_hbm.at[idx_c.at[pl.ds(i * 128, 128)]],
                rows_v.at[pl.ds(i * 128, 128)], sem))
        for d in descs:
            d.wait()
        pltpu.sync_copy(rows_v, out_hbm.at[pl.ds(off, GGROWS)])
        return 0
    lax.fori_loop(0, ROWS_W // GGROWS, group_body, 0)

    ntail = ROWS_W % GGROWS
    if ntail:
        off = wbase + (ROWS_W // GGROWS) * GGROWS
        pltpu.sync_copy(idx_hbm.at[pl.ds(off, ntail)], idx_c.at[pl.ds(0, ntail)])
        pltpu.async_copy(
            tbl_hbm.at[idx_c.at[pl.ds(0, ntail)]],
            rows_v.at[pl.ds(0, ntail)], sem).wait()
        pltpu.sync_copy(rows_v.at[pl.ds(0, ntail)], out_hbm.at[pl.ds(off, ntail)])


def _sc_gather_e64_body(tbl_hbm, idx_hbm, out_hbm, idx_c, rows_v, sem):
    wid = lax.axis_index("s") * NC + lax.axis_index("c")
    _gather_worker(tbl_hbm, idx_hbm, out_hbm, idx_c, rows_v, sem, wid * ROWS_W)


def _sc_gather_e64(tbl, idx):
    mesh = plsc.VectorSubcoreMesh(core_axis_name="c", subcore_axis_name="s")
    f = pl.kernel(
        _sc_gather_e64_body,
        out_type=jax.ShapeDtypeStruct((E, DE), jnp.float32),
        mesh=mesh,
        compiler_params=pltpu.CompilerParams(
            use_tc_tiling_on_sc=False, needs_layout_passes=False),
        scratch_types=[
            pltpu.VMEM((GGROWS,), jnp.int32),
            pltpu.VMEM((GGROWS, DE), jnp.float32),
            pltpu.SemaphoreType.DMA,
        ],
    )
    return f(tbl, idx)


def _sc_dual_gather_body(t1_hbm, t2_hbm, i1_hbm, i2_hbm, o1_hbm, o2_hbm,
                         idx_c, rows_v, sem):
    wid = lax.axis_index("s") * NC + lax.axis_index("c")
    wbase = wid * ROWS_W
    _gather_worker(t1_hbm, i1_hbm, o1_hbm, idx_c, rows_v, sem, wbase)
    _gather_worker(t2_hbm, i2_hbm, o2_hbm, idx_c, rows_v, sem, wbase)


def _sc_dual_gather(t1, t2, i1, i2):
    mesh = plsc.VectorSubcoreMesh(core_axis_name="c", subcore_axis_name="s")
    f = pl.kernel(
        _sc_dual_gather_body,
        out_type=[jax.ShapeDtypeStruct((E, DE), jnp.float32),
                  jax.ShapeDtypeStruct((E, DE), jnp.float32)],
        mesh=mesh,
        compiler_params=pltpu.CompilerParams(
            use_tc_tiling_on_sc=False, needs_layout_passes=False),
        scratch_types=[
            pltpu.VMEM((GGROWS,), jnp.int32),
            pltpu.VMEM((GGROWS, DE), jnp.float32),
            pltpu.SemaphoreType.DMA,
        ],
    )
    return f(t1, t2, i1, i2)


def kernel(h, m_st, rbf, cbf0, cbf1, idx_s, idx_t, idx_swap, id3_kt, id3_st, id3_ragged_idx,
           W_ca, W_mkt, W_rbf3, W_down, W_cbf, W_st3, W_ts3, rb_w1, rb_w2, ra_w1, ra_w2,
           W_rbf_h, W_atom, at_w1, at_w2, W_self, rs_w1, rs_w2):
    m_kt = _tc1(m_st, rbf, W_mkt, W_rbf3, W_down)

    mkt_pad = jnp.concatenate(
        [m_kt, jnp.zeros((PADROWS, DT), jnp.float32)], axis=0)
    m2f = _sc_route(id3_st.astype(jnp.int32), id3_ragged_idx.astype(jnp.int32),
                    id3_kt.astype(jnp.int32), mkt_pad).reshape(E, KMAX * DT)

    Wf = W_cbf.transpose(1, 0, 2).reshape(DCBF * DT, DT)
    P, Q, R, Tm = _routing_consts()
    x_st, xts_pre = _tc3(m2f, cbf1.reshape(E, KMAX * NSPH), cbf0.reshape(E, DCBF * NSPH),
                         P, Q, R, Tm, Wf, W_st3, W_ts3)
    xts_g = _sc_gather_e64(xts_pre, idx_swap.astype(jnp.int32))

    m_mid, a = _tc5(x_st, xts_g, m_st, rbf, W_ca, rb_w1, rb_w2, ra_w1, ra_w2, W_rbf_h)

    seg = jax.ops.segment_sum(a, idx_t, num_segments=N)

    Ws1 = W_self[0:DA]
    Ws2 = W_self[DA:2 * DA]
    Ws3 = W_self[2 * DA:]
    h_new, hs1, hs2 = _tc7(seg, jnp.zeros_like(seg), h, W_atom, at_w1, at_w2, Ws1, Ws2)

    g1, g2 = _sc_dual_gather(hs1, hs2, idx_s.astype(jnp.int32),
                             idx_t.astype(jnp.int32))
    m_out = _tc9(g1, g2, m_mid, Ws3, rs_w1, rs_w2)
    return (h_new, m_out)


# speedup vs baseline: 1.1191x; 1.1191x over previous
import functools

import jax
import jax.numpy as jnp
import numpy as np
from jax import lax
from jax.experimental import pallas as pl
from jax.experimental.pallas import tpu as pltpu
from jax.experimental.pallas import tpu_sc as plsc

N = 10000
E = 640000
T = 1280000
KMAX = 4
NSPH = 7
DA = 128
DE = 64
DRBF = 16
DCBF = 16
DT = 32
INV_SQRT_2 = float(1.0 / np.sqrt(2.0))


def _silu(x):
    return x * jax.nn.sigmoid(x)


def _tc1_body(m_st_ref, rbf_ref, wmkt_ref, wrbf3_ref, wdown_ref, out_ref):
    mkt = _silu(m_st_ref[...] @ wmkt_ref[...])
    mkt = mkt * (rbf_ref[...] @ wrbf3_ref[...])
    out_ref[...] = _silu(mkt @ wdown_ref[...])


def _tc1(m_st, rbf, W_mkt, W_rbf3, W_down, be=8000):
    grid = (E // be,)
    return pl.pallas_call(
        _tc1_body,
        grid=grid,
        in_specs=[
            pl.BlockSpec((be, DE), lambda i: (i, 0)),
            pl.BlockSpec((be, DRBF), lambda i: (i, 0)),
            pl.BlockSpec((DE, DE), lambda i: (0, 0)),
            pl.BlockSpec((DRBF, DE), lambda i: (0, 0)),
            pl.BlockSpec((DE, DT), lambda i: (0, 0)),
        ],
        out_specs=pl.BlockSpec((be, DT), lambda i: (i, 0)),
        out_shape=jax.ShapeDtypeStruct((E, DT), jnp.float32),
    )(m_st, rbf, W_mkt, W_rbf3, W_down)


SC_LANES = NSPH * DT
RW_LANES = DCBF * DT


def _routing_consts():
    P = np.zeros((KMAX * NSPH, KMAX * SC_LANES), np.float32)
    Q = np.zeros((KMAX * DT, KMAX * SC_LANES), np.float32)
    for k in range(KMAX):
        for s in range(NSPH):
            for c in range(DT):
                P[k * NSPH + s, k * SC_LANES + s * DT + c] = 1.0
                Q[k * DT + c, k * SC_LANES + s * DT + c] = 1.0
    R = np.zeros((NSPH * DCBF * NSPH, RW_LANES), np.float32)
    Tm = np.zeros((NSPH * SC_LANES, RW_LANES), np.float32)
    for s in range(NSPH):
        for i in range(DCBF):
            for c in range(DT):
                R[s * (DCBF * NSPH) + i * NSPH + s, i * DT + c] = 1.0
                Tm[s * SC_LANES + s * DT + c, i * DT + c] = 1.0
    return jnp.asarray(P), jnp.asarray(Q), jnp.asarray(R), jnp.asarray(Tm)


def _tc3_body(m2f_ref, cbf1f_ref, cbf0f_ref, p_ref, q_ref, r_ref, t_ref,
              wf_ref, wst3_ref, wts3_ref, xst_ref, xts_ref):
    m2f = m2f_ref[...]
    cbf1f = cbf1f_ref[...]
    cbf0f = cbf0f_ref[...]

    cp = cbf1f @ p_ref[...]
    qp = m2f @ q_ref[...]
    sk = cp[:, :SC_LANES] * qp[:, :SC_LANES]
    for k in range(1, KMAX):
        sk += cp[:, k * SC_LANES:(k + 1) * SC_LANES] * qp[:, k * SC_LANES:(k + 1) * SC_LANES]

    rw = None
    for s in range(NSPH):
        r_s = cbf0f @ r_ref[s * (DCBF * NSPH):(s + 1) * (DCBF * NSPH), :]
        t_s = sk @ t_ref[s * SC_LANES:(s + 1) * SC_LANES, :]
        term = r_s * t_s
        rw = term if rw is None else rw + term

    x = rw @ wf_ref[...]
    xst_ref[...] = _silu(x @ wst3_ref[...])
    xts_ref[...] = _silu(x @ wts3_ref[...])


def _tc3(m2f, cbf1f, cbf0f, P, Q, R, Tm, Wf, W_st3, W_ts3, be=640):
    grid = (E // be,)
    out_shape = [
        jax.ShapeDtypeStruct((E, DE), jnp.float32),
        jax.ShapeDtypeStruct((E, DE), jnp.float32),
    ]
    return pl.pallas_call(
        _tc3_body,
        grid=grid,
        in_specs=[
            pl.BlockSpec((be, KMAX * DT), lambda i: (i, 0)),
            pl.BlockSpec((be, KMAX * NSPH), lambda i: (i, 0)),
            pl.BlockSpec((be, DCBF * NSPH), lambda i: (i, 0)),
            pl.BlockSpec(P.shape, lambda i: (0, 0)),
            pl.BlockSpec(Q.shape, lambda i: (0, 0)),
            pl.BlockSpec(R.shape, lambda i: (0, 0)),
            pl.BlockSpec(Tm.shape, lambda i: (0, 0)),
            pl.BlockSpec((RW_LANES, DT), lambda i: (0, 0)),
            pl.BlockSpec((DT, DE), lambda i: (0, 0)),
            pl.BlockSpec((DT, DE), lambda i: (0, 0)),
        ],
        out_specs=[
            pl.BlockSpec((be, DE), lambda i: (i, 0)),
            pl.BlockSpec((be, DE), lambda i: (i, 0)),
        ],
        out_shape=out_shape,
    )(m2f, cbf1f, cbf0f, P, Q, R, Tm, Wf, W_st3, W_ts3)


def _tc5_body(xst_ref, xts_ref, m_st_ref, rbf_ref, wca_ref, rbw1_ref, rbw2_ref,
              raw1_ref, raw2_ref, wrbfh_ref, m_ref, aout_ref):
    x_ca = _silu(m_st_ref[...] @ wca_ref[...])
    x = x_ca * INV_SQRT_2 + (xst_ref[...] + xts_ref[...]) * 0.5
    y = _silu(x @ rbw1_ref[...])
    y = _silu(y @ rbw2_ref[...])
    x = (x + y) * INV_SQRT_2
    m = (m_st_ref[...] + x) * INV_SQRT_2
    y = _silu(m @ raw1_ref[...])
    y = _silu(y @ raw2_ref[...])
    m = (m + y) * INV_SQRT_2
    m_ref[...] = m
    aout_ref[...] = m * (rbf_ref[...] @ wrbfh_ref[...])


def _tc5(x_st, xts_g, m_st, rbf, W_ca, rb_w1, rb_w2, ra_w1, ra_w2, W_rbf_h, be=8000):
    grid = (E // be,)
    return pl.pallas_call(
        _tc5_body,
        grid=grid,
        in_specs=[
            pl.BlockSpec((be, DE), lambda i: (i, 0)),
            pl.BlockSpec((be, DE), lambda i: (i, 0)),
            pl.BlockSpec((be, DE), lambda i: (i, 0)),
            pl.BlockSpec((be, DRBF), lambda i: (i, 0)),
            pl.BlockSpec((DE, DE), lambda i: (0, 0)),
            pl.BlockSpec((DE, DE), lambda i: (0, 0)),
            pl.BlockSpec((DE, DE), lambda i: (0, 0)),
            pl.BlockSpec((DE, DE), lambda i: (0, 0)),
            pl.BlockSpec((DE, DE), lambda i: (0, 0)),
            pl.BlockSpec((DRBF, DE), lambda i: (0, 0)),
        ],
        out_specs=[
            pl.BlockSpec((be, DE), lambda i: (i, 0)),
            pl.BlockSpec((be, DE), lambda i: (i, 0)),
        ],
        out_shape=[
            jax.ShapeDtypeStruct((E, DE), jnp.float32),
            jax.ShapeDtypeStruct((E, DE), jnp.float32),
        ],
    )(x_st, xts_g, m_st, rbf, W_ca, rb_w1, rb_w2, ra_w1, ra_w2, W_rbf_h)


def _tc7_body(p0_ref, p1_ref, h_ref, watom_ref, atw1_ref, atw2_ref,
              ws1_ref, ws2_ref, hnew_ref, hs1_ref, hs2_ref):
    x = p0_ref[...] + p1_ref[...]
    x = _silu(x @ watom_ref[...])
    y = _silu(x @ atw1_ref[...])
    y = _silu(y @ atw2_ref[...])
    x = (x + y) * INV_SQRT_2
    h_new = (h_ref[...] + x) * INV_SQRT_2
    hnew_ref[...] = h_new
    hs1_ref[...] = h_new @ ws1_ref[...]
    hs2_ref[...] = h_new @ ws2_ref[...]


def _tc7(p0, p1, h, W_atom, at_w1, at_w2, Ws1, Ws2, bn=2000):
    grid = (N // bn,)
    return pl.pallas_call(
        _tc7_body,
        grid=grid,
        in_specs=[
            pl.BlockSpec((bn, DE), lambda i: (i, 0)),
            pl.BlockSpec((bn, DE), lambda i: (i, 0)),
            pl.BlockSpec((bn, DA), lambda i: (i, 0)),
            pl.BlockSpec((DE, DA), lambda i: (0, 0)),
            pl.BlockSpec((DA, DA), lambda i: (0, 0)),
            pl.BlockSpec((DA, DA), lambda i: (0, 0)),
            pl.BlockSpec((DA, DE), lambda i: (0, 0)),
            pl.BlockSpec((DA, DE), lambda i: (0, 0)),
        ],
        out_specs=[
            pl.BlockSpec((bn, DA), lambda i: (i, 0)),
            pl.BlockSpec((bn, DE), lambda i: (i, 0)),
            pl.BlockSpec((bn, DE), lambda i: (i, 0)),
        ],
        out_shape=[
            jax.ShapeDtypeStruct((N, DA), jnp.float32),
            jax.ShapeDtypeStruct((N, DE), jnp.float32),
            jax.ShapeDtypeStruct((N, DE), jnp.float32),
        ],
    )(p0, p1, h, W_atom, at_w1, at_w2, Ws1, Ws2)


def _tc9_body(g1_ref, g2_ref, m_ref, ws3_ref, rsw1_ref, rsw2_ref, out_ref):
    m = m_ref[...]
    y = _silu(g1_ref[...] + g2_ref[...] + m @ ws3_ref[...])
    m = (m + y) * INV_SQRT_2
    z = _silu(m @ rsw1_ref[...])
    z = _silu(z @ rsw2_ref[...])
    out_ref[...] = (m + z) * INV_SQRT_2


def _tc9(g1, g2, m_mid, Ws3, rs_w1, rs_w2, be=8000):
    grid = (E // be,)
    return pl.pallas_call(
        _tc9_body,
        grid=grid,
        in_specs=[
            pl.BlockSpec((be, DE), lambda i: (i, 0)),
            pl.BlockSpec((be, DE), lambda i: (i, 0)),
            pl.BlockSpec((be, DE), lambda i: (i, 0)),
            pl.BlockSpec((DE, DE), lambda i: (0, 0)),
            pl.BlockSpec((DE, DE), lambda i: (0, 0)),
            pl.BlockSpec((DE, DE), lambda i: (0, 0)),
        ],
        out_specs=pl.BlockSpec((be, DE), lambda i: (i, 0)),
        out_shape=jax.ShapeDtypeStruct((E, DE), jnp.float32),
    )(g1, g2, m_mid, Ws3, rs_w1, rs_w2)


NW = 32
NC = 2
SLOTS = E * KMAX
SLOTS_W = SLOTS // NW
PADROWS = 64
CH3 = 2048
GK = 5
GSTREAM = 80
GROWS = GK * GSTREAM


def _sc_route_body(st_hbm, rg_hbm, kt_hbm, mkt_hbm, m2_hbm,
                   src_v, st_c, rg_c, kt_c, rows_v, sem, semw):
    wid = lax.axis_index("s") * NC + lax.axis_index("c")
    base_slot = wid * SLOTS_W
    lane = lax.iota(jnp.int32, 16)
    intmax = jnp.int32(2**31 - 1)
    nchunks = T // CH3

    def init_body(j, _):
        src_v[pl.ds(j * 16, 16)] = E + ((j * 16 + lane) & (PADROWS - 1))
        return 0
    lax.fori_loop(0, SLOTS_W // 16, init_body, 0)

    def prefetch(cidx, par):
        pltpu.async_copy(st_hbm.at[pl.ds(cidx * CH3, CH3)],
                         st_c.at[pl.ds(par * CH3, CH3)], sem)
        pltpu.async_copy(rg_hbm.at[pl.ds(cidx * CH3, CH3)],
                         rg_c.at[pl.ds(par * CH3, CH3)], sem)
        pltpu.async_copy(kt_hbm.at[pl.ds(cidx * CH3, CH3)],
                         kt_c.at[pl.ds(par * CH3, CH3)], sem)

    def drain(par):
        for buf in (st_c, rg_c, kt_c):
            pltpu.make_async_copy(st_hbm.at[pl.ds(0, CH3)],
                                  buf.at[pl.ds(par * CH3, CH3)], sem).wait()

    prefetch(0, 0)

    def chunk_body(cidx, _):
        par = lax.rem(cidx, 2)
        drain(par)

        @pl.when(cidx + 1 < nchunks)
        def _():
            prefetch(cidx + 1, 1 - par)

        def scan_one(v16):
            st = st_c[pl.ds(par * CH3 + v16, 16)]
            rg = rg_c[pl.ds(par * CH3 + v16, 16)]
            kt = kt_c[pl.ds(par * CH3 + v16, 16)]
            loc = st * KMAX + rg - base_slot
            inr = (loc >= 0) & (loc < SLOTS_W)

            @pl.when(jnp.max(jnp.where(inr, 1, 0)) > 0)
            def _():
                comp = jnp.where(inr, loc * 16 + lane, intmax)
                ck, cv = plsc.sort_key_val(comp, kt)
                slot = lax.shift_right_arithmetic(ck, 4)
                nxt = ck.at[jnp.minimum(lane + 1, 15)].get(
                    mode="promise_in_bounds")
                win = (ck != intmax) & ((slot != lax.shift_right_arithmetic(nxt, 4))
                                        | (lane == 15))
                plsc.store_scatter(src_v, [jnp.where(win, slot, 0)], cv, mask=win)

        def vreg_body(v, _):
            scan_one(v * 32)
            scan_one(v * 32 + 16)
            return 0
        lax.fori_loop(0, CH3 // 32, vreg_body, 0)
        return 0
    lax.fori_loop(0, nchunks, chunk_body, 0)

    ngroups = SLOTS_W // GROWS

    def gather_body(g, _):
        par = lax.rem(g, 2)
        off = g * GROWS

        @pl.when(g >= 2)
        def _():
            pltpu.make_async_copy(
                rows_v.at[pl.ds(par * GROWS, GROWS)],
                m2_hbm.at[pl.ds(base_slot + (g - 2) * GROWS, GROWS)],
                semw).wait()

        descs = []
        for i in range(GK):
            descs.append(pltpu.async_copy(
                mkt_hbm.at[src_v.at[pl.ds(off + i * GSTREAM, GSTREAM)]],
                rows_v.at[pl.ds(par * GROWS + i * GSTREAM, GSTREAM)], sem))
        for d in descs:
            d.wait()
        pltpu.async_copy(rows_v.at[pl.ds(par * GROWS, GROWS)],
                         m2_hbm.at[pl.ds(base_slot + off, GROWS)], semw)
        return 0
    lax.fori_loop(0, ngroups, gather_body, 0)
    for gl in (ngroups - 2, ngroups - 1):
        pltpu.make_async_copy(
            rows_v.at[pl.ds(lax.rem(gl, 2) * GROWS, GROWS)],
            m2_hbm.at[pl.ds(base_slot + gl * GROWS, GROWS)], semw).wait()


def _sc_route(id3_st, id3_rg, id3_kt, mkt_pad):
    mesh = plsc.VectorSubcoreMesh(core_axis_name="c", subcore_axis_name="s")
    f = pl.kernel(
        _sc_route_body,
        out_type=jax.ShapeDtypeStruct((SLOTS, DT), jnp.float32),
        mesh=mesh,
        compiler_params=pltpu.CompilerParams(use_tc_tiling_on_sc=False, needs_layout_passes=False),
        scratch_types=[
            pltpu.VMEM((SLOTS_W,), jnp.int32),
            pltpu.VMEM((2 * CH3,), jnp.int32),
            pltpu.VMEM((2 * CH3,), jnp.int32),
            pltpu.VMEM((2 * CH3,), jnp.int32),
            pltpu.VMEM((2 * GROWS, DT), jnp.float32),
            pltpu.SemaphoreType.DMA,
            pltpu.SemaphoreType.DMA,
        ],
    )
    return f(id3_st, id3_rg, id3_kt, mkt_pad)


ROWS_W = E // NW
GG = 4
GGROWS = GG * 128
NFULL = ROWS_W // 128
TAIL = ROWS_W - NFULL * 128


def _gather_worker(tbl_hbm, idx_hbm, out_hbm, idx_c, rows_v, sem, wbase):
    def group_body(g, _):
        off = wbase + g * GGROWS
        pltpu.sync_copy(idx_hbm.at[pl.ds(off, GGROWS)], idx_c)
        descs = []
        for i in range(GG):
            descs.append(pltpu.async_copy(
                tbl_hbm.at[idx_c.at[pl.ds(i * 128, 128)]],
                rows_v.at[pl.ds(i * 128, 128)], sem))
        for d in descs:
            d.wait()
        pltpu.sync_copy(rows_v, out_hbm.at[pl.ds(off, GGROWS)])
        return 0
    lax.fori_loop(0, ROWS_W // GGROWS, group_body, 0)

    ntail = ROWS_W % GGROWS
    if ntail:
        off = wbase + (ROWS_W // GGROWS) * GGROWS
        pltpu.sync_copy(idx_hbm.at[pl.ds(off, ntail)], idx_c.at[pl.ds(0, ntail)])
        pltpu.async_copy(
            tbl_hbm.at[idx_c.at[pl.ds(0, ntail)]],
            rows_v.at[pl.ds(0, ntail)], sem).wait()
        pltpu.sync_copy(rows_v.at[pl.ds(0, ntail)], out_hbm.at[pl.ds(off, ntail)])


def _sc_gather_e64_body(tbl_hbm, idx_hbm, out_hbm, idx_c, rows_v, sem):
    wid = lax.axis_index("s") * NC + lax.axis_index("c")
    _gather_worker(tbl_hbm, idx_hbm, out_hbm, idx_c, rows_v, sem, wid * ROWS_W)


def _sc_gather_e64(tbl, idx):
    mesh = plsc.VectorSubcoreMesh(core_axis_name="c", subcore_axis_name="s")
    f = pl.kernel(
        _sc_gather_e64_body,
        out_type=jax.ShapeDtypeStruct((E, DE), jnp.float32),
        mesh=mesh,
        compiler_params=pltpu.CompilerParams(
            use_tc_tiling_on_sc=False, needs_layout_passes=False),
        scratch_types=[
            pltpu.VMEM((GGROWS,), jnp.int32),
            pltpu.VMEM((GGROWS, DE), jnp.float32),
            pltpu.SemaphoreType.DMA,
        ],
    )
    return f(tbl, idx)


def _sc_dual_gather_body(t1_hbm, t2_hbm, i1_hbm, i2_hbm, o1_hbm, o2_hbm,
                         idx_c, rows_v, sem):
    wid = lax.axis_index("s") * NC + lax.axis_index("c")
    wbase = wid * ROWS_W
    _gather_worker(t1_hbm, i1_hbm, o1_hbm, idx_c, rows_v, sem, wbase)
    _gather_worker(t2_hbm, i2_hbm, o2_hbm, idx_c, rows_v, sem, wbase)


def _sc_dual_gather(t1, t2, i1, i2):
    mesh = plsc.VectorSubcoreMesh(core_axis_name="c", subcore_axis_name="s")
    f = pl.kernel(
        _sc_dual_gather_body,
        out_type=[jax.ShapeDtypeStruct((E, DE), jnp.float32),
                  jax.ShapeDtypeStruct((E, DE), jnp.float32)],
        mesh=mesh,
        compiler_params=pltpu.CompilerParams(
            use_tc_tiling_on_sc=False, needs_layout_passes=False),
        scratch_types=[
            pltpu.VMEM((GGROWS,), jnp.int32),
            pltpu.VMEM((GGROWS, DE), jnp.float32),
            pltpu.SemaphoreType.DMA,
        ],
    )
    return f(t1, t2, i1, i2)


SEG_CH = 400


def _sc_segsum_body(a_hbm, idx_hbm, zero_hbm, out_hbm, idx_c, rows_v, acc_sh, sem):
    cid = lax.axis_index("c")
    sid = lax.axis_index("s")
    wid = sid * NC + cid

    @pl.when(sid == 0)
    def _():
        pltpu.sync_copy(zero_hbm, acc_sh)
    plsc.subcore_barrier()

    wbase = wid * ROWS_W

    def body(g, _):
        off = wbase + g * SEG_CH
        pltpu.sync_copy(idx_hbm.at[pl.ds(off, SEG_CH)], idx_c)
        pltpu.sync_copy(a_hbm.at[pl.ds(off, SEG_CH)], rows_v)
        pltpu.sync_copy(rows_v, acc_sh.at[idx_c], add=True)
        return 0
    lax.fori_loop(0, ROWS_W // SEG_CH, body, 0)
    plsc.subcore_barrier()

    @pl.when(sid < 10)
    def _():
        pltpu.sync_copy(acc_sh.at[pl.ds(sid * 1000, 1000)],
                        out_hbm.at[pl.ds(cid * N + sid * 1000, 1000)])


def _sc_segsum(a, idx):
    mesh = plsc.VectorSubcoreMesh(core_axis_name="c", subcore_axis_name="s")
    f = pl.kernel(
        _sc_segsum_body,
        out_type=jax.ShapeDtypeStruct((2 * N, DE), jnp.float32),
        mesh=mesh,
        compiler_params=pltpu.CompilerParams(
            use_tc_tiling_on_sc=False, needs_layout_passes=False),
        scratch_types=[
            pltpu.VMEM((SEG_CH,), jnp.int32),
            pltpu.VMEM((SEG_CH, DE), jnp.float32),
            pltpu.VMEM_SHARED((N, DE), jnp.float32),
            pltpu.SemaphoreType.DMA,
        ],
    )
    return f(a, idx, jnp.zeros((N, DE), jnp.float32))


def kernel(h, m_st, rbf, cbf0, cbf1, idx_s, idx_t, idx_swap, id3_kt, id3_st, id3_ragged_idx,
           W_ca, W_mkt, W_rbf3, W_down, W_cbf, W_st3, W_ts3, rb_w1, rb_w2, ra_w1, ra_w2,
           W_rbf_h, W_atom, at_w1, at_w2, W_self, rs_w1, rs_w2):
    m_kt = _tc1(m_st, rbf, W_mkt, W_rbf3, W_down)

    mkt_pad = jnp.concatenate(
        [m_kt, jnp.zeros((PADROWS, DT), jnp.float32)], axis=0)
    m2f = _sc_route(id3_st.astype(jnp.int32), id3_ragged_idx.astype(jnp.int32),
                    id3_kt.astype(jnp.int32), mkt_pad).reshape(E, KMAX * DT)

    Wf = W_cbf.transpose(1, 0, 2).reshape(DCBF * DT, DT)
    P, Q, R, Tm = _routing_consts()
    x_st, xts_pre = _tc3(m2f, cbf1.reshape(E, KMAX * NSPH), cbf0.reshape(E, DCBF * NSPH),
                         P, Q, R, Tm, Wf, W_st3, W_ts3)
    xts_g = _sc_gather_e64(xts_pre, idx_swap.astype(jnp.int32))

    m_mid, a = _tc5(x_st, xts_g, m_st, rbf, W_ca, rb_w1, rb_w2, ra_w1, ra_w2, W_rbf_h)

    parts = _sc_segsum(a, idx_t.astype(jnp.int32))

    Ws1 = W_self[0:DA]
    Ws2 = W_self[DA:2 * DA]
    Ws3 = W_self[2 * DA:]
    h_new, hs1, hs2 = _tc7(parts[:N], parts[N:], h, W_atom, at_w1, at_w2, Ws1, Ws2)

    g1, g2 = _sc_dual_gather(hs1, hs2, idx_s.astype(jnp.int32),
                             idx_t.astype(jnp.int32))
    m_out = _tc9(g1, g2, m_mid, Ws3, rs_w1, rs_w2)
    return (h_new, m_out)


# speedup vs baseline: 1.1213x; 1.0020x over previous
import functools

import jax
import jax.numpy as jnp
import numpy as np
from jax import lax
from jax.experimental import pallas as pl
from jax.experimental.pallas import tpu as pltpu
from jax.experimental.pallas import tpu_sc as plsc

N = 10000
E = 640000
T = 1280000
KMAX = 4
NSPH = 7
DA = 128
DE = 64
DRBF = 16
DCBF = 16
DT = 32
INV_SQRT_2 = float(1.0 / np.sqrt(2.0))


def _silu(x):
    return x * jax.nn.sigmoid(x)


def _tc1_body(m_st_ref, rbf_ref, wmkt_ref, wrbf3_ref, wdown_ref, out_ref):
    mkt = _silu(m_st_ref[...] @ wmkt_ref[...])
    mkt = mkt * (rbf_ref[...] @ wrbf3_ref[...])
    out_ref[...] = _silu(mkt @ wdown_ref[...])


def _tc1(m_st, rbf, W_mkt, W_rbf3, W_down, be=8000):
    grid = (E // be,)
    return pl.pallas_call(
        _tc1_body,
        grid=grid,
        in_specs=[
            pl.BlockSpec((be, DE), lambda i: (i, 0)),
            pl.BlockSpec((be, DRBF), lambda i: (i, 0)),
            pl.BlockSpec((DE, DE), lambda i: (0, 0)),
            pl.BlockSpec((DRBF, DE), lambda i: (0, 0)),
            pl.BlockSpec((DE, DT), lambda i: (0, 0)),
        ],
        out_specs=pl.BlockSpec((be, DT), lambda i: (i, 0)),
        out_shape=jax.ShapeDtypeStruct((E, DT), jnp.float32),
    )(m_st, rbf, W_mkt, W_rbf3, W_down)


SC_LANES = NSPH * DT
RW_LANES = DCBF * DT


def _routing_consts():
    P = np.zeros((KMAX * NSPH, KMAX * SC_LANES), np.float32)
    Q = np.zeros((KMAX * DT, KMAX * SC_LANES), np.float32)
    for k in range(KMAX):
        for s in range(NSPH):
            for c in range(DT):
                P[k * NSPH + s, k * SC_LANES + s * DT + c] = 1.0
                Q[k * DT + c, k * SC_LANES + s * DT + c] = 1.0
    R = np.zeros((NSPH * DCBF * NSPH, RW_LANES), np.float32)
    Tm = np.zeros((NSPH * SC_LANES, RW_LANES), np.float32)
    for s in range(NSPH):
        for i in range(DCBF):
            for c in range(DT):
                R[s * (DCBF * NSPH) + i * NSPH + s, i * DT + c] = 1.0
                Tm[s * SC_LANES + s * DT + c, i * DT + c] = 1.0
    return jnp.asarray(P), jnp.asarray(Q), jnp.asarray(R), jnp.asarray(Tm)


def _tc3_body(m2f_ref, cbf1f_ref, cbf0f_ref, p_ref, q_ref, r_ref, t_ref,
              wf_ref, wst3_ref, wts3_ref, xst_ref, xts_ref):
    m2f = m2f_ref[...]
    cbf1f = cbf1f_ref[...]
    cbf0f = cbf0f_ref[...]

    cp = cbf1f @ p_ref[...]
    qp = m2f @ q_ref[...]
    sk = cp[:, :SC_LANES] * qp[:, :SC_LANES]
    for k in range(1, KMAX):
        sk += cp[:, k * SC_LANES:(k + 1) * SC_LANES] * qp[:, k * SC_LANES:(k + 1) * SC_LANES]

    rw = None
    for s in range(NSPH):
        r_s = cbf0f @ r_ref[s * (DCBF * NSPH):(s + 1) * (DCBF * NSPH), :]
        t_s = sk @ t_ref[s * SC_LANES:(s + 1) * SC_LANES, :]
        term = r_s * t_s
        rw = term if rw is None else rw + term

    x = rw @ wf_ref[...]
    xst_ref[...] = _silu(x @ wst3_ref[...])
    xts_ref[...] = _silu(x @ wts3_ref[...])


def _tc3(m2f, cbf1f, cbf0f, P, Q, R, Tm, Wf, W_st3, W_ts3, be=640):
    grid = (E // be,)
    out_shape = [
        jax.ShapeDtypeStruct((E, DE), jnp.float32),
        jax.ShapeDtypeStruct((E, DE), jnp.float32),
    ]
    return pl.pallas_call(
        _tc3_body,
        grid=grid,
        in_specs=[
            pl.BlockSpec((be, KMAX * DT), lambda i: (i, 0)),
            pl.BlockSpec((be, KMAX * NSPH), lambda i: (i, 0)),
            pl.BlockSpec((be, DCBF * NSPH), lambda i: (i, 0)),
            pl.BlockSpec(P.shape, lambda i: (0, 0)),
            pl.BlockSpec(Q.shape, lambda i: (0, 0)),
            pl.BlockSpec(R.shape, lambda i: (0, 0)),
            pl.BlockSpec(Tm.shape, lambda i: (0, 0)),
            pl.BlockSpec((RW_LANES, DT), lambda i: (0, 0)),
            pl.BlockSpec((DT, DE), lambda i: (0, 0)),
            pl.BlockSpec((DT, DE), lambda i: (0, 0)),
        ],
        out_specs=[
            pl.BlockSpec((be, DE), lambda i: (i, 0)),
            pl.BlockSpec((be, DE), lambda i: (i, 0)),
        ],
        out_shape=out_shape,
    )(m2f, cbf1f, cbf0f, P, Q, R, Tm, Wf, W_st3, W_ts3)


def _tc5_body(xst_ref, xts_ref, m_st_ref, rbf_ref, wca_ref, rbw1_ref, rbw2_ref,
              raw1_ref, raw2_ref, wrbfh_ref, m_ref, aout_ref):
    x_ca = _silu(m_st_ref[...] @ wca_ref[...])
    x = x_ca * INV_SQRT_2 + (xst_ref[...] + xts_ref[...]) * 0.5
    y = _silu(x @ rbw1_ref[...])
    y = _silu(y @ rbw2_ref[...])
    x = (x + y) * INV_SQRT_2
    m = (m_st_ref[...] + x) * INV_SQRT_2
    y = _silu(m @ raw1_ref[...])
    y = _silu(y @ raw2_ref[...])
    m = (m + y) * INV_SQRT_2
    m_ref[...] = m
    aout_ref[...] = m * (rbf_ref[...] @ wrbfh_ref[...])


def _tc5(x_st, xts_g, m_st, rbf, W_ca, rb_w1, rb_w2, ra_w1, ra_w2, W_rbf_h, be=8000):
    grid = (E // be,)
    return pl.pallas_call(
        _tc5_body,
        grid=grid,
        in_specs=[
            pl.BlockSpec((be, DE), lambda i: (i, 0)),
            pl.BlockSpec((be, DE), lambda i: (i, 0)),
            pl.BlockSpec((be, DE), lambda i: (i, 0)),
            pl.BlockSpec((be, DRBF), lambda i: (i, 0)),
            pl.BlockSpec((DE, DE), lambda i: (0, 0)),
            pl.BlockSpec((DE, DE), lambda i: (0, 0)),
            pl.BlockSpec((DE, DE), lambda i: (0, 0)),
            pl.BlockSpec((DE, DE), lambda i: (0, 0)),
            pl.BlockSpec((DE, DE), lambda i: (0, 0)),
            pl.BlockSpec((DRBF, DE), lambda i: (0, 0)),
        ],
        out_specs=[
            pl.BlockSpec((be, DE), lambda i: (i, 0)),
            pl.BlockSpec((be, DE), lambda i: (i, 0)),
        ],
        out_shape=[
            jax.ShapeDtypeStruct((E, DE), jnp.float32),
            jax.ShapeDtypeStruct((E, DE), jnp.float32),
        ],
    )(x_st, xts_g, m_st, rbf, W_ca, rb_w1, rb_w2, ra_w1, ra_w2, W_rbf_h)


def _tc7_body(p0_ref, p1_ref, h_ref, watom_ref, atw1_ref, atw2_ref,
              ws1_ref, ws2_ref, hnew_ref, hs1_ref, hs2_ref):
    x = p0_ref[...] + p1_ref[...]
    x = _silu(x @ watom_ref[...])
    y = _silu(x @ atw1_ref[...])
    y = _silu(y @ atw2_ref[...])
    x = (x + y) * INV_SQRT_2
    h_new = (h_ref[...] + x) * INV_SQRT_2
    hnew_ref[...] = h_new
    hs1_ref[...] = h_new @ ws1_ref[...]
    hs2_ref[...] = h_new @ ws2_ref[...]


def _tc7(p0, p1, h, W_atom, at_w1, at_w2, Ws1, Ws2, bn=2000):
    grid = (N // bn,)
    return pl.pallas_call(
        _tc7_body,
        grid=grid,
        in_specs=[
            pl.BlockSpec((bn, DE), lambda i: (i, 0)),
            pl.BlockSpec((bn, DE), lambda i: (i, 0)),
            pl.BlockSpec((bn, DA), lambda i: (i, 0)),
            pl.BlockSpec((DE, DA), lambda i: (0, 0)),
            pl.BlockSpec((DA, DA), lambda i: (0, 0)),
            pl.BlockSpec((DA, DA), lambda i: (0, 0)),
            pl.BlockSpec((DA, DE), lambda i: (0, 0)),
            pl.BlockSpec((DA, DE), lambda i: (0, 0)),
        ],
        out_specs=[
            pl.BlockSpec((bn, DA), lambda i: (i, 0)),
            pl.BlockSpec((bn, DE), lambda i: (i, 0)),
            pl.BlockSpec((bn, DE), lambda i: (i, 0)),
        ],
        out_shape=[
            jax.ShapeDtypeStruct((N, DA), jnp.float32),
            jax.ShapeDtypeStruct((N, DE), jnp.float32),
            jax.ShapeDtypeStruct((N, DE), jnp.float32),
        ],
    )(p0, p1, h, W_atom, at_w1, at_w2, Ws1, Ws2)


def _tc9_body(g1_ref, g2_ref, m_ref, ws3_ref, rsw1_ref, rsw2_ref, out_ref):
    m = m_ref[...]
    y = _silu(g1_ref[...] + g2_ref[...] + m @ ws3_ref[...])
    m = (m + y) * INV_SQRT_2
    z = _silu(m @ rsw1_ref[...])
    z = _silu(z @ rsw2_ref[...])
    out_ref[...] = (m + z) * INV_SQRT_2


def _tc9(g1, g2, m_mid, Ws3, rs_w1, rs_w2, be=8000):
    grid = (E // be,)
    return pl.pallas_call(
        _tc9_body,
        grid=grid,
        in_specs=[
            pl.BlockSpec((be, DE), lambda i: (i, 0)),
            pl.BlockSpec((be, DE), lambda i: (i, 0)),
            pl.BlockSpec((be, DE), lambda i: (i, 0)),
            pl.BlockSpec((DE, DE), lambda i: (0, 0)),
            pl.BlockSpec((DE, DE), lambda i: (0, 0)),
            pl.BlockSpec((DE, DE), lambda i: (0, 0)),
        ],
        out_specs=pl.BlockSpec((be, DE), lambda i: (i, 0)),
        out_shape=jax.ShapeDtypeStruct((E, DE), jnp.float32),
    )(g1, g2, m_mid, Ws3, rs_w1, rs_w2)


NW = 32
NC = 2
SLOTS = E * KMAX
SLOTS_W = SLOTS // NW
PADROWS = 64
CH3 = 2048
GK = 5
GSTREAM = 80
GROWS = GK * GSTREAM


def _sc_route_body(st_hbm, rg_hbm, kt_hbm, mkt_hbm, m2_hbm,
                   src_v, st_c, rg_c, kt_c, rows_v, sem, semw):
    wid = lax.axis_index("s") * NC + lax.axis_index("c")
    base_slot = wid * SLOTS_W
    lane = lax.iota(jnp.int32, 16)
    intmax = jnp.int32(2**31 - 1)
    nchunks = T // CH3

    def init_body(j, _):
        src_v[pl.ds(j * 16, 16)] = E + ((j * 16 + lane) & (PADROWS - 1))
        return 0
    lax.fori_loop(0, SLOTS_W // 16, init_body, 0)

    def prefetch(cidx, par):
        pltpu.async_copy(st_hbm.at[pl.ds(cidx * CH3, CH3)],
                         st_c.at[pl.ds(par * CH3, CH3)], sem)
        pltpu.async_copy(rg_hbm.at[pl.ds(cidx * CH3, CH3)],
                         rg_c.at[pl.ds(par * CH3, CH3)], sem)
        pltpu.async_copy(kt_hbm.at[pl.ds(cidx * CH3, CH3)],
                         kt_c.at[pl.ds(par * CH3, CH3)], sem)

    def drain(par):
        for buf in (st_c, rg_c, kt_c):
            pltpu.make_async_copy(st_hbm.at[pl.ds(0, CH3)],
                                  buf.at[pl.ds(par * CH3, CH3)], sem).wait()

    prefetch(0, 0)

    def chunk_body(cidx, _):
        par = lax.rem(cidx, 2)
        drain(par)

        @pl.when(cidx + 1 < nchunks)
        def _():
            prefetch(cidx + 1, 1 - par)

        def scan_one(v16):
            st = st_c[pl.ds(par * CH3 + v16, 16)]
            rg = rg_c[pl.ds(par * CH3 + v16, 16)]
            kt = kt_c[pl.ds(par * CH3 + v16, 16)]
            loc = st * KMAX + rg - base_slot
            inr = (loc >= 0) & (loc < SLOTS_W)

            @pl.when(jnp.max(jnp.where(inr, 1, 0)) > 0)
            def _():
                comp = jnp.where(inr, loc * 16 + lane, intmax)
                ck, cv = plsc.sort_key_val(comp, kt)
                slot = lax.shift_right_arithmetic(ck, 4)
                nxt = ck.at[jnp.minimum(lane + 1, 15)].get(
                    mode="promise_in_bounds")
                win = (ck != intmax) & ((slot != lax.shift_right_arithmetic(nxt, 4))
                                        | (lane == 15))
                plsc.store_scatter(src_v, [jnp.where(win, slot, 0)], cv, mask=win)

        def vreg_body(v, _):
            scan_one(v * 64)
            scan_one(v * 64 + 16)
            scan_one(v * 64 + 32)
            scan_one(v * 64 + 48)
            return 0
        lax.fori_loop(0, CH3 // 64, vreg_body, 0)
        return 0
    lax.fori_loop(0, nchunks, chunk_body, 0)

    ngroups = SLOTS_W // GROWS

    def gather_body(g, _):
        par = lax.rem(g, 2)
        off = g * GROWS

        @pl.when(g >= 2)
        def _():
            pltpu.make_async_copy(
                rows_v.at[pl.ds(par * GROWS, GROWS)],
                m2_hbm.at[pl.ds(base_slot + (g - 2) * GROWS, GROWS)],
                semw).wait()

        descs = []
        for i in range(GK):
            descs.append(pltpu.async_copy(
                mkt_hbm.at[src_v.at[pl.ds(off + i * GSTREAM, GSTREAM)]],
                rows_v.at[pl.ds(par * GROWS + i * GSTREAM, GSTREAM)], sem))
        for d in descs:
            d.wait()
        pltpu.async_copy(rows_v.at[pl.ds(par * GROWS, GROWS)],
                         m2_hbm.at[pl.ds(base_slot + off, GROWS)], semw)
        return 0
    lax.fori_loop(0, ngroups, gather_body, 0)
    for gl in (ngroups - 2, ngroups - 1):
        pltpu.make_async_copy(
            rows_v.at[pl.ds(lax.rem(gl, 2) * GROWS, GROWS)],
            m2_hbm.at[pl.ds(base_slot + gl * GROWS, GROWS)], semw).wait()


def _sc_route(id3_st, id3_rg, id3_kt, mkt_pad):
    mesh = plsc.VectorSubcoreMesh(core_axis_name="c", subcore_axis_name="s")
    f = pl.kernel(
        _sc_route_body,
        out_type=jax.ShapeDtypeStruct((SLOTS, DT), jnp.float32),
        mesh=mesh,
        compiler_params=pltpu.CompilerParams(use_tc_tiling_on_sc=False, needs_layout_passes=False),
        scratch_types=[
            pltpu.VMEM((SLOTS_W,), jnp.int32),
            pltpu.VMEM((2 * CH3,), jnp.int32),
            pltpu.VMEM((2 * CH3,), jnp.int32),
            pltpu.VMEM((2 * CH3,), jnp.int32),
            pltpu.VMEM((2 * GROWS, DT), jnp.float32),
            pltpu.SemaphoreType.DMA,
            pltpu.SemaphoreType.DMA,
        ],
    )
    return f(id3_st, id3_rg, id3_kt, mkt_pad)


ROWS_W = E // NW
GG = 4
GGROWS = GG * 128
NFULL = ROWS_W // 128
TAIL = ROWS_W - NFULL * 128


def _gather_worker(tbl_hbm, idx_hbm, out_hbm, idx_c, rows_v, sem, wbase):
    def group_body(g, _):
        off = wbase + g * GGROWS
        pltpu.sync_copy(idx_hbm.at[pl.ds(off, GGROWS)], idx_c)
        descs = []
        for i in range(GG):
            descs.append(pltpu.async_copy(
                tbl_hbm.at[idx_c.at[pl.ds(i * 128, 128)]],
                rows_v.at[pl.ds(i * 128, 128)], sem))
        for d in descs:
            d.wait()
        pltpu.sync_copy(rows_v, out_hbm.at[pl.ds(off, GGROWS)])
        return 0
    lax.fori_loop(0, ROWS_W // GGROWS, group_body, 0)

    ntail = ROWS_W % GGROWS
    if ntail:
        off = wbase + (ROWS_W // GGROWS) * GGROWS
        pltpu.sync_copy(idx_hbm.at[pl.ds(off, ntail)], idx_c.at[pl.ds(0, ntail)])
        pltpu.async_copy(
            tbl_hbm.at[idx_c.at[pl.ds(0, ntail)]],
            rows_v.at[pl.ds(0, ntail)], sem).wait()
        pltpu.sync_copy(rows_v.at[pl.ds(0, ntail)], out_hbm.at[pl.ds(off, ntail)])


def _sc_gather_e64_body(tbl_hbm, idx_hbm, out_hbm, idx_c, rows_v, sem):
    wid = lax.axis_index("s") * NC + lax.axis_index("c")
    _gather_worker(tbl_hbm, idx_hbm, out_hbm, idx_c, rows_v, sem, wid * ROWS_W)


def _sc_gather_e64(tbl, idx):
    mesh = plsc.VectorSubcoreMesh(core_axis_name="c", subcore_axis_name="s")
    f = pl.kernel(
        _sc_gather_e64_body,
        out_type=jax.ShapeDtypeStruct((E, DE), jnp.float32),
        mesh=mesh,
        compiler_params=pltpu.CompilerParams(
            use_tc_tiling_on_sc=False, needs_layout_passes=False),
        scratch_types=[
            pltpu.VMEM((GGROWS,), jnp.int32),
            pltpu.VMEM((GGROWS, DE), jnp.float32),
            pltpu.SemaphoreType.DMA,
        ],
    )
    return f(tbl, idx)


def _sc_dual_gather_body(t1_hbm, t2_hbm, i1_hbm, i2_hbm, o1_hbm, o2_hbm,
                         idx_c, rows_v, sem):
    wid = lax.axis_index("s") * NC + lax.axis_index("c")
    wbase = wid * ROWS_W
    _gather_worker(t1_hbm, i1_hbm, o1_hbm, idx_c, rows_v, sem, wbase)
    _gather_worker(t2_hbm, i2_hbm, o2_hbm, idx_c, rows_v, sem, wbase)


def _sc_dual_gather(t1, t2, i1, i2):
    mesh = plsc.VectorSubcoreMesh(core_axis_name="c", subcore_axis_name="s")
    f = pl.kernel(
        _sc_dual_gather_body,
        out_type=[jax.ShapeDtypeStruct((E, DE), jnp.float32),
                  jax.ShapeDtypeStruct((E, DE), jnp.float32)],
        mesh=mesh,
        compiler_params=pltpu.CompilerParams(
            use_tc_tiling_on_sc=False, needs_layout_passes=False),
        scratch_types=[
            pltpu.VMEM((GGROWS,), jnp.int32),
            pltpu.VMEM((GGROWS, DE), jnp.float32),
            pltpu.SemaphoreType.DMA,
        ],
    )
    return f(t1, t2, i1, i2)


SEG_CH = 400


def _sc_segsum_body(a_hbm, idx_hbm, zero_hbm, out_hbm, idx_c, rows_v, acc_sh, sem):
    cid = lax.axis_index("c")
    sid = lax.axis_index("s")
    wid = sid * NC + cid

    @pl.when(sid == 0)
    def _():
        pltpu.sync_copy(zero_hbm, acc_sh)
    plsc.subcore_barrier()

    wbase = wid * ROWS_W

    def body(g, _):
        off = wbase + g * SEG_CH
        pltpu.sync_copy(idx_hbm.at[pl.ds(off, SEG_CH)], idx_c)
        pltpu.sync_copy(a_hbm.at[pl.ds(off, SEG_CH)], rows_v)
        pltpu.sync_copy(rows_v, acc_sh.at[idx_c], add=True)
        return 0
    lax.fori_loop(0, ROWS_W // SEG_CH, body, 0)
    plsc.subcore_barrier()

    @pl.when(sid < 10)
    def _():
        pltpu.sync_copy(acc_sh.at[pl.ds(sid * 1000, 1000)],
                        out_hbm.at[pl.ds(cid * N + sid * 1000, 1000)])


def _sc_segsum(a, idx):
    mesh = plsc.VectorSubcoreMesh(core_axis_name="c", subcore_axis_name="s")
    f = pl.kernel(
        _sc_segsum_body,
        out_type=jax.ShapeDtypeStruct((2 * N, DE), jnp.float32),
        mesh=mesh,
        compiler_params=pltpu.CompilerParams(
            use_tc_tiling_on_sc=False, needs_layout_passes=False),
        scratch_types=[
            pltpu.VMEM((SEG_CH,), jnp.int32),
            pltpu.VMEM((SEG_CH, DE), jnp.float32),
            pltpu.VMEM_SHARED((N, DE), jnp.float32),
            pltpu.SemaphoreType.DMA,
        ],
    )
    return f(a, idx, jnp.zeros((N, DE), jnp.float32))


def kernel(h, m_st, rbf, cbf0, cbf1, idx_s, idx_t, idx_swap, id3_kt, id3_st, id3_ragged_idx,
           W_ca, W_mkt, W_rbf3, W_down, W_cbf, W_st3, W_ts3, rb_w1, rb_w2, ra_w1, ra_w2,
           W_rbf_h, W_atom, at_w1, at_w2, W_self, rs_w1, rs_w2):
    m_kt = _tc1(m_st, rbf, W_mkt, W_rbf3, W_down)

    mkt_pad = jnp.concatenate(
        [m_kt, jnp.zeros((PADROWS, DT), jnp.float32)], axis=0)
    m2f = _sc_route(id3_st.astype(jnp.int32), id3_ragged_idx.astype(jnp.int32),
                    id3_kt.astype(jnp.int32), mkt_pad).reshape(E, KMAX * DT)

    Wf = W_cbf.transpose(1, 0, 2).reshape(DCBF * DT, DT)
    P, Q, R, Tm = _routing_consts()
    x_st, xts_pre = _tc3(m2f, cbf1.reshape(E, KMAX * NSPH), cbf0.reshape(E, DCBF * NSPH),
                         P, Q, R, Tm, Wf, W_st3, W_ts3)
    xts_g = _sc_gather_e64(xts_pre, idx_swap.astype(jnp.int32))

    m_mid, a = _tc5(x_st, xts_g, m_st, rbf, W_ca, rb_w1, rb_w2, ra_w1, ra_w2, W_rbf_h)

    parts = _sc_segsum(a, idx_t.astype(jnp.int32))

    Ws1 = W_self[0:DA]
    Ws2 = W_self[DA:2 * DA]
    Ws3 = W_self[2 * DA:]
    h_new, hs1, hs2 = _tc7(parts[:N], parts[N:], h, W_atom, at_w1, at_w2, Ws1, Ws2)

    g1, g2 = _sc_dual_gather(hs1, hs2, idx_s.astype(jnp.int32),
                             idx_t.astype(jnp.int32))
    m_out = _tc9(g1, g2, m_mid, Ws3, rs_w1, rs_w2)
    return (h_new, m_out)
